# baseline (speedup 1.0000x reference)
import jax
import jax.numpy as jnp
from jax.experimental import pallas as pl
from jax.experimental.pallas import tpu as pltpu

NUM_TOKENS = 22
TT = 32
SMALL = 8
HIDDEN = 64
RPB = 32


def _fuse_body(table_ref, wt_ref, b_ref, f_ref):
    f_ref[...] = (
        jnp.dot(table_ref[...], wt_ref[...], preferred_element_type=jnp.float32)
        + b_ref[...]
    )


def _emb_body(srcT_ref, f_ref, out_ref):
    f = f_ref[...]
    t_iota = jax.lax.broadcasted_iota(jnp.int32, (1, TT), 1)
    for r in range(RPB):
        col = srcT_ref[0, :, r:r + 1]
        oh = (col == t_iota).astype(jnp.float32)
        out_ref[r] = jnp.dot(oh, f, preferred_element_type=jnp.float32)


def kernel(src, table, W, b):
    B, S = src.shape
    n_tok = B * S
    n_rows = n_tok // 128
    grid = n_rows // RPB
    table_pad = jnp.zeros((TT, SMALL), jnp.float32).at[:NUM_TOKENS].set(table)
    wt = W.T
    b2 = b.reshape(1, HIDDEN)

    F = pl.pallas_call(
        _fuse_body,
        out_shape=jax.ShapeDtypeStruct((TT, HIDDEN), jnp.float32),
    )(table_pad, wt, b2)

    srcT = src.reshape(grid, RPB, 128).transpose(0, 2, 1)

    out3 = pl.pallas_call(
        _emb_body,
        grid=(grid,),
        in_specs=[
            pl.BlockSpec((1, 128, RPB), lambda i: (i, 0, 0)),
            pl.BlockSpec((TT, HIDDEN), lambda i: (0, 0)),
        ],
        out_specs=pl.BlockSpec((RPB, 128, HIDDEN), lambda i: (i, 0, 0)),
        out_shape=jax.ShapeDtypeStruct((n_rows, 128, HIDDEN), jnp.float32),
    )(srcT, F)
    return out3.reshape(B, S, HIDDEN)


# speedup vs baseline: 1.2324x; 1.2324x over previous
import jax
import jax.numpy as jnp
from jax.experimental import pallas as pl
from jax.experimental.pallas import tpu as pltpu

NUM_TOKENS = 22
TT = 32
SMALL = 8
HIDDEN = 64
RPB = 128


def _fuse_body(table_ref, wt_ref, b_ref, f_ref):
    f_ref[...] = (
        jnp.dot(table_ref[...], wt_ref[...], preferred_element_type=jnp.float32)
        + b_ref[...]
    )


def _emb_body(srcT_ref, f_ref, out_ref):
    f = f_ref[...]
    t_iota = jax.lax.broadcasted_iota(jnp.int32, (1, TT), 1)
    for r in range(RPB):
        col = srcT_ref[0, :, r:r + 1]
        oh = (col == t_iota).astype(jnp.float32)
        out_ref[r] = jnp.dot(oh, f, preferred_element_type=jnp.float32)


def kernel(src, table, W, b):
    B, S = src.shape
    n_tok = B * S
    n_rows = n_tok // 128
    grid = n_rows // RPB
    table_pad = jnp.zeros((TT, SMALL), jnp.float32).at[:NUM_TOKENS].set(table)
    wt = W.T
    b2 = b.reshape(1, HIDDEN)

    F = pl.pallas_call(
        _fuse_body,
        out_shape=jax.ShapeDtypeStruct((TT, HIDDEN), jnp.float32),
    )(table_pad, wt, b2)

    srcT = src.reshape(grid, RPB, 128).transpose(0, 2, 1)

    out3 = pl.pallas_call(
        _emb_body,
        grid=(grid,),
        in_specs=[
            pl.BlockSpec((1, 128, RPB), lambda i: (i, 0, 0)),
            pl.BlockSpec((TT, HIDDEN), lambda i: (0, 0)),
        ],
        out_specs=pl.BlockSpec((RPB, 128, HIDDEN), lambda i: (i, 0, 0)),
        out_shape=jax.ShapeDtypeStruct((n_rows, 128, HIDDEN), jnp.float32),
    )(srcT, F)
    return out3.reshape(B, S, HIDDEN)


# speedup vs baseline: 1.2616x; 1.0237x over previous
import jax
import jax.numpy as jnp
from jax.experimental import pallas as pl
from jax.experimental.pallas import tpu as pltpu

NUM_TOKENS = 22
TT = 32
SMALL = 8
HIDDEN = 64
RPB = 256


def _fuse_body(table_ref, wt_ref, b_ref, f_ref):
    f_ref[...] = (
        jnp.dot(table_ref[...], wt_ref[...], preferred_element_type=jnp.float32)
        + b_ref[...]
    )


def _emb_body(srcT_ref, f_ref, out_ref):
    f = f_ref[...]
    t_iota = jax.lax.broadcasted_iota(jnp.int32, (1, TT), 1)
    for r in range(RPB):
        col = srcT_ref[0, :, r:r + 1]
        oh = (col == t_iota).astype(jnp.float32)
        out_ref[r] = jnp.dot(oh, f, preferred_element_type=jnp.float32)


def kernel(src, table, W, b):
    B, S = src.shape
    n_tok = B * S
    n_rows = n_tok // 128
    grid = n_rows // RPB
    table_pad = jnp.zeros((TT, SMALL), jnp.float32).at[:NUM_TOKENS].set(table)
    wt = W.T
    b2 = b.reshape(1, HIDDEN)

    F = pl.pallas_call(
        _fuse_body,
        out_shape=jax.ShapeDtypeStruct((TT, HIDDEN), jnp.float32),
    )(table_pad, wt, b2)

    srcT = src.reshape(grid, RPB, 128).transpose(0, 2, 1)

    out3 = pl.pallas_call(
        _emb_body,
        grid=(grid,),
        in_specs=[
            pl.BlockSpec((1, 128, RPB), lambda i: (i, 0, 0)),
            pl.BlockSpec((TT, HIDDEN), lambda i: (0, 0)),
        ],
        out_specs=pl.BlockSpec((RPB, 128, HIDDEN), lambda i: (i, 0, 0)),
        out_shape=jax.ShapeDtypeStruct((n_rows, 128, HIDDEN), jnp.float32),
    )(srcT, F)
    return out3.reshape(B, S, HIDDEN)
